# Initial kernel scaffold; baseline (speedup 1.0000x reference)
import jax
import jax.numpy as jnp
from jax.experimental import pallas as pl


def kernel(pts, weight_volume, volume_bounds):
    raise NotImplementedError("write your pallas kernel here")


# speedup vs baseline: 1.6560x; 1.6560x over previous
import functools

import jax
import jax.numpy as jnp
from jax import lax
from jax.experimental import pallas as pl
from jax.experimental.pallas import tpu as pltpu
from jax.experimental.pallas import tpu_sc as plsc

CH = 55
CPAD = 64
P = 128
G = P // 16
NW = 32
OUTW = P * CH


def _tec_kernel(nchunks, table, xs, ys, zs, consts, out,
                idx_v, w_v, rows_v, x_v, y_v, z_v, c_v, out_v, sem):
    wid = lax.axis_index("s") * 2 + lax.axis_index("c")
    nt = (nchunks - wid + NW - 1) // NW
    pltpu.sync_copy(consts, c_v)

    def chunk_body(t, carry):
        cid = wid + t * NW
        base = cid * P
        pltpu.sync_copy(xs.at[pl.ds(base, P)], x_v)
        pltpu.sync_copy(ys.at[pl.ds(base, P)], y_v)
        pltpu.sync_copy(zs.at[pl.ds(base, P)], z_v)

        def grp_index(g, c2):
            xv = x_v[pl.ds(g * 16, 16)]
            yv = y_v[pl.ds(g * 16, 16)]
            zv = z_v[pl.ds(g * 16, 16)]
            cd = jnp.clip(xv * c_v[0, :] + c_v[3, :], 0.0, 63.0)
            ch = jnp.clip(yv * c_v[1, :] + c_v[4, :], 0.0, 63.0)
            cw = jnp.clip(zv * c_v[2, :] + c_v[5, :], 0.0, 63.0)
            d0 = cd.astype(jnp.int32)
            h0 = ch.astype(jnp.int32)
            w0 = cw.astype(jnp.int32)
            fd = cd - d0.astype(jnp.float32)
            fh = ch - h0.astype(jnp.float32)
            fw = cw - w0.astype(jnp.float32)
            one = jnp.float32(1.0)
            gd, gh, gw = one - fd, one - fh, one - fw
            d1 = jnp.minimum(d0 + 1, 63)
            h1 = jnp.minimum(h0 + 1, 63)
            w1 = jnp.minimum(w0 + 1, 63)
            bd0 = d0 * 4096
            bd1 = d1 * 4096
            bh0 = h0 * 64
            bh1 = h1 * 64
            i00 = bd0 + bh0
            i01 = bd0 + bh1
            i10 = bd1 + bh0
            i11 = bd1 + bh1
            sl = pl.ds(g * 16, 16)
            idx_v[0, sl] = i00 + w0
            idx_v[1, sl] = i00 + w1
            idx_v[2, sl] = i01 + w0
            idx_v[3, sl] = i01 + w1
            idx_v[4, sl] = i10 + w0
            idx_v[5, sl] = i10 + w1
            idx_v[6, sl] = i11 + w0
            idx_v[7, sl] = i11 + w1
            hgw = gh * gw
            hgf = gh * fw
            hfg = fh * gw
            hff = fh * fw
            w_v[0, sl] = gd * hgw
            w_v[1, sl] = gd * hgf
            w_v[2, sl] = gd * hfg
            w_v[3, sl] = gd * hff
            w_v[4, sl] = fd * hgw
            w_v[5, sl] = fd * hgf
            w_v[6, sl] = fd * hfg
            w_v[7, sl] = fd * hff
            return c2

        lax.fori_loop(0, G, grp_index, 0)

        handles = [
            pltpu.async_copy(table.at[idx_v.at[j]], rows_v.at[j], sem)
            for j in range(8)
        ]
        for h in handles:
            h.wait()

        def grp_blend(g, c2):
            wrows = [w_v[j, pl.ds(g * 16, 16)] for j in range(8)]
            for p in range(16):
                pt = g * 16 + p
                acc = [None] * 4
                for j in range(8):
                    wsp = jnp.broadcast_to(wrows[j][p], (16,))
                    for k in range(4):
                        r = rows_v[j, pt, pl.ds(k * 16, 16)]
                        if acc[k] is None:
                            acc[k] = wsp * r
                        else:
                            acc[k] = acc[k] + wsp * r
                for k in range(4):
                    out_v[pl.ds(pt * CH + k * 16, 16)] = acc[k]
            return c2

        lax.fori_loop(0, G, grp_blend, 0)
        pltpu.sync_copy(out_v.at[pl.ds(0, OUTW)],
                        out.at[pl.ds(cid * OUTW, OUTW)])
        return carry

    lax.fori_loop(0, nt, chunk_body, 0)


@functools.partial(jax.jit, static_argnums=(5,))
def _run(table, xs, ys, zs, consts, npts):
    nchunks = npts // P
    mesh = plsc.VectorSubcoreMesh(core_axis_name="c", subcore_axis_name="s")
    kern = functools.partial(
        pl.kernel,
        out_type=jax.ShapeDtypeStruct((npts * CH,), jnp.float32),
        mesh=mesh,
        compiler_params=pltpu.CompilerParams(use_tc_tiling_on_sc=False),
        scratch_types=[
            pltpu.VMEM((8, P), jnp.int32),
            pltpu.VMEM((8, P), jnp.float32),
            pltpu.VMEM((8, P, CPAD), jnp.float32),
            pltpu.VMEM((P,), jnp.float32),
            pltpu.VMEM((P,), jnp.float32),
            pltpu.VMEM((P,), jnp.float32),
            pltpu.VMEM((6, 16), jnp.float32),
            pltpu.VMEM((OUTW + 16,), jnp.float32),
            pltpu.SemaphoreType.DMA,
        ],
    )(functools.partial(_tec_kernel, nchunks))
    return kern(table, xs, ys, zs, consts)


def kernel(pts, weight_volume, volume_bounds):
    B, N, _ = pts.shape
    vol = weight_volume[0]
    C, D, H, W = vol.shape
    table = jnp.pad(vol.reshape(C, -1).T, ((0, 0), (0, CPAD - C)))
    ptsf = pts.reshape(-1, 3)
    xs = ptsf[:, 0]
    ys = ptsf[:, 1]
    zs = ptsf[:, 2]
    vb0 = volume_bounds[0]
    vlen = volume_bounds[1] - volume_bounds[0]
    dims = jnp.array([D - 1, H - 1, W - 1], dtype=jnp.float32)
    scale = dims / vlen
    off = -vb0 * scale
    consts = jnp.broadcast_to(
        jnp.concatenate([scale, off])[:, None], (6, 16)).astype(jnp.float32)
    out = _run(table, xs, ys, zs, consts, B * N)
    return out[: B * N * CH].reshape(B, N, CH)


# speedup vs baseline: 1.6775x; 1.0129x over previous
import functools

import jax
import jax.numpy as jnp
from jax import lax
from jax.experimental import pallas as pl
from jax.experimental.pallas import tpu as pltpu
from jax.experimental.pallas import tpu_sc as plsc

CH = 55
CPAD = 64
P = 128
G = P // 16
NW = 32
OUTW = P * CH
def _tec_kernel(nchunks, table, ptsf, consts, out,
                idx_v, w_v, rows_v, p_v, c_v, out_v,
                sem0, sem1, osem0, osem1):
    wid = lax.axis_index("s") * 2 + lax.axis_index("c")
    nt = (nchunks - wid + NW - 1) // NW
    pltpu.sync_copy(consts, c_v)
    sems = (sem0, sem1)

    def fire(b, t):
        cid = wid + t * NW
        base = cid * P
        pltpu.sync_copy(ptsf.at[0, pl.ds(base, P)], p_v.at[0])
        pltpu.sync_copy(ptsf.at[1, pl.ds(base, P)], p_v.at[1])
        pltpu.sync_copy(ptsf.at[2, pl.ds(base, P)], p_v.at[2])

        def grp_index(g, c2):
            sl16 = pl.ds(g * 16, 16)
            xv = p_v[0, sl16]
            yv = p_v[1, sl16]
            zv = p_v[2, sl16]
            cd = jnp.clip(xv * c_v[0, :] + c_v[3, :], 0.0, 63.0)
            chh = jnp.clip(yv * c_v[1, :] + c_v[4, :], 0.0, 63.0)
            cw = jnp.clip(zv * c_v[2, :] + c_v[5, :], 0.0, 63.0)
            d0 = cd.astype(jnp.int32)
            h0 = chh.astype(jnp.int32)
            w0 = cw.astype(jnp.int32)
            fd = cd - d0.astype(jnp.float32)
            fh = chh - h0.astype(jnp.float32)
            fw = cw - w0.astype(jnp.float32)
            one = jnp.float32(1.0)
            gd, gh, gw = one - fd, one - fh, one - fw
            d1 = jnp.minimum(d0 + 1, 63)
            h1 = jnp.minimum(h0 + 1, 63)
            w1 = jnp.minimum(w0 + 1, 63)
            bd0 = d0 * 4096
            bd1 = d1 * 4096
            bh0 = h0 * 64
            bh1 = h1 * 64
            i00 = bd0 + bh0
            i01 = bd0 + bh1
            i10 = bd1 + bh0
            i11 = bd1 + bh1
            sl = pl.ds(g * 16, 16)
            idx_v[b, 0, sl] = i00 + w0
            idx_v[b, 1, sl] = i00 + w1
            idx_v[b, 2, sl] = i01 + w0
            idx_v[b, 3, sl] = i01 + w1
            idx_v[b, 4, sl] = i10 + w0
            idx_v[b, 5, sl] = i10 + w1
            idx_v[b, 6, sl] = i11 + w0
            idx_v[b, 7, sl] = i11 + w1
            hgw = gh * gw
            hgf = gh * fw
            hfg = fh * gw
            hff = fh * fw
            w_v[b, 0, sl] = gd * hgw
            w_v[b, 1, sl] = gd * hgf
            w_v[b, 2, sl] = gd * hfg
            w_v[b, 3, sl] = gd * hff
            w_v[b, 4, sl] = fd * hgw
            w_v[b, 5, sl] = fd * hgf
            w_v[b, 6, sl] = fd * hfg
            w_v[b, 7, sl] = fd * hff
            return c2

        lax.fori_loop(0, G, grp_index, 0)
        for j in range(8):
            pltpu.async_copy(table.at[idx_v.at[b, j]], rows_v.at[b, j],
                             sems[b])

    def wait_rows(b):
        for j in range(8):
            pltpu.make_async_copy(table.at[pl.ds(0, P)], rows_v.at[b, j],
                                  sems[b]).wait()

    osems = (osem0, osem1)

    def drain_out(b):
        pltpu.make_async_copy(out_v.at[b, pl.ds(0, OUTW)],
                              out.at[pl.ds(0, OUTW)], osems[b]).wait()

    himask = jnp.full((16,), -65536, dtype=jnp.int32)

    def blend(b, t):
        def grp_blend(g, c2):
            wrows = [w_v[b, j, pl.ds(g * 16, 16)] for j in range(8)]
            for p in range(16):
                pt = g * 16 + p
                acc = [None] * 4
                for j in range(8):
                    wsp = jnp.broadcast_to(wrows[j][p], (16,))
                    w0i = rows_v[b, j, pt, pl.ds(0, 16)]
                    w1i = rows_v[b, j, pt, pl.ds(16, 16)]
                    corner = (
                        lax.bitcast_convert_type(
                            lax.shift_left(w0i, 16), jnp.float32),
                        lax.bitcast_convert_type(w0i & himask, jnp.float32),
                        lax.bitcast_convert_type(
                            lax.shift_left(w1i, 16), jnp.float32),
                        lax.bitcast_convert_type(w1i & himask, jnp.float32),
                    )
                    for k in range(4):
                        if acc[k] is None:
                            acc[k] = wsp * corner[k]
                        else:
                            acc[k] = acc[k] + wsp * corner[k]
                for k in range(4):
                    out_v[b, pl.ds(pt * CH + 16 * k, 16)] = acc[k]
            return c2

        lax.fori_loop(0, G, grp_blend, 0)
        cid = wid + t * NW
        pltpu.async_copy(out_v.at[b, pl.ds(0, OUTW)],
                         out.at[pl.ds(cid * OUTW, OUTW)], osems[b])

    fire(0, 0)

    def body(tt, carry):
        t0 = tt * 2

        @pl.when(t0 + 1 < nt)
        def _():
            fire(1, t0 + 1)

        @pl.when(t0 >= 2)
        def _():
            drain_out(0)

        wait_rows(0)
        blend(0, t0)

        @pl.when(t0 + 1 < nt)
        def _():
            @pl.when(t0 + 2 < nt)
            def _():
                fire(0, t0 + 2)

            @pl.when(t0 >= 1)
            def _():
                drain_out(1)

            wait_rows(1)
            blend(1, t0 + 1)

        return carry

    lax.fori_loop(0, (nt + 1) // 2, body, 0)
    drain_out(0)
    drain_out(1)


@functools.partial(jax.jit, static_argnums=(3,))
def _run(table, ptsf, consts, npts):
    nchunks = npts // P
    mesh = plsc.VectorSubcoreMesh(core_axis_name="c", subcore_axis_name="s")
    kern = functools.partial(
        pl.kernel,
        out_type=jax.ShapeDtypeStruct((npts * CH,), jnp.float32),
        mesh=mesh,
        compiler_params=pltpu.CompilerParams(use_tc_tiling_on_sc=False),
        scratch_types=[
            pltpu.VMEM((2, 8, P), jnp.int32),
            pltpu.VMEM((2, 8, P), jnp.float32),
            pltpu.VMEM((2, 8, P, CPAD // 2), jnp.int32),
            pltpu.VMEM((3, P), jnp.float32),
            pltpu.VMEM((6, 16), jnp.float32),
            pltpu.VMEM((2, OUTW + 16), jnp.float32),
            pltpu.SemaphoreType.DMA,
            pltpu.SemaphoreType.DMA,
            pltpu.SemaphoreType.DMA,
            pltpu.SemaphoreType.DMA,
        ],
    )(functools.partial(_tec_kernel, nchunks))
    return kern(table, ptsf, consts)


def kernel(pts, weight_volume, volume_bounds):
    B, N, _ = pts.shape
    vol = weight_volume[0]
    C, D, H, W = vol.shape
    volT = jnp.pad(vol.reshape(C, -1).T, ((0, 0), (0, CPAD - C)))
    perm = [32 * (p // 32) + 16 * (p % 2) + (p % 32) // 2 for p in range(CPAD)]
    tb16 = volT[:, jnp.array(perm)].astype(jnp.bfloat16)
    table = lax.bitcast_convert_type(
        tb16.reshape(-1, CPAD // 2, 2), jnp.int32)
    ptsf = pts.reshape(-1, 3).T
    vb0 = volume_bounds[0]
    vlen = volume_bounds[1] - volume_bounds[0]
    dims = jnp.array([D - 1, H - 1, W - 1], dtype=jnp.float32)
    scale = dims / vlen
    off = -vb0 * scale
    consts = jnp.broadcast_to(
        jnp.concatenate([scale, off])[:, None], (6, 16)).astype(jnp.float32)
    out = _run(table, ptsf, consts, B * N)
    return out.reshape(B, N, CH)
